# Initial kernel scaffold; baseline (speedup 1.0000x reference)
import jax
import jax.numpy as jnp
from jax.experimental import pallas as pl


def kernel(atom_features, pair_features, pair_split, pair_membership, atom_split, atom_to_pair, W_AA, b_AA, W_PA, b_PA, W_A, b_A, W_AP, b_AP, W_PP, b_PP, W_P, b_P):
    raise NotImplementedError("write your pallas kernel here")


# speedup vs baseline: 3.5353x; 3.5353x over previous
import functools

import jax
import jax.numpy as jnp
from jax import lax
from jax.experimental import pallas as pl
from jax.experimental.pallas import tpu as pltpu
from jax.experimental.pallas import tpu_sc as plsc

NA = 50000
NP = 800000
DA = 75
DP = 14
H = 50

NC = 2
NS = 16
NW = NC * NS
BPG = 64
NBG = NP // BPG
BPS = 128

NPASS = 4
PBASE = (0, 12512, 25024, 37536)
PSIZE = (12512, 12512, 12512, 12464)
RPT = 392
LROWS = (360, 360, 360, 312)

BA = 2000
NAB = NA // BA
BPR = 8000
NPB = NP // BPR

_f32 = jnp.float32


def _relu(x):
    return jnp.maximum(x, 0.0)


def _atoms_body(af, wuv, bvec, waa, baa, t, aa):
    x = af[...]
    t[...] = jnp.dot(x, wuv[...], preferred_element_type=_f32) + bvec[...]
    aa[...] = _relu(jnp.dot(x, waa[...], preferred_element_type=_f32) + baa[...])


def _pa_body(pf, wpa, bpa, pa):
    pa[...] = _relu(jnp.dot(pf[...], wpa[...], preferred_element_type=_f32)
                    + bpa[...])


def _p_body(s, pf, wp1, wpp, bpp, wp2, bp, p):
    pp = _relu(jnp.dot(pf[...], wpp[...], preferred_element_type=_f32) + bpp[...])
    x = (jnp.dot(s[...], wp1[...], preferred_element_type=_f32)
         + jnp.dot(pp, wp2[...], preferred_element_type=_f32) + bp[...])
    p[...] = _relu(x)


def _a_body(aa, seg, waa2, wseg, ba, a):
    x = (jnp.dot(aa[...], waa2[...], preferred_element_type=_f32)
         + jnp.dot(seg[...], wseg[...], preferred_element_type=_f32) + ba[...])
    a[...] = _relu(x)


_sc_mesh = plsc.VectorSubcoreMesh(core_axis_name="c", subcore_axis_name="s")


@functools.partial(
    pl.kernel,
    out_type=(
        jax.ShapeDtypeStruct((NP, 64), _f32),
        jax.ShapeDtypeStruct((NA, 64), _f32),
    ),
    mesh=_sc_mesh,
    scratch_types=(
        pltpu.VMEM((BPG,), jnp.int32),
        pltpu.VMEM((BPG,), jnp.int32),
        pltpu.VMEM((BPG, 128), _f32),
        pltpu.VMEM((BPG, 128), _f32),
        pltpu.VMEM((BPG, 64), _f32),
        pltpu.VMEM((BPS + 16,), jnp.int32),
        pltpu.VMEM((BPS, 64), _f32),
        pltpu.VMEM((144,), jnp.int32),
        pltpu.VMEM((RPT, 64), _f32),
        pltpu.SemaphoreType.DMA,
        pltpu.SemaphoreType.DMA,
    ),
)
def _sc_stage(t_hbm, ii_hbm, jj_hbm, pa_hbm, ids_hbm, bnd_hbm,
              s_hbm, seg_hbm,
              bi_v, bj_v, ti_v, tj_v, s_v, idv, pa_v, bnd_v, acc,
              sem1, sem2):
    c = lax.axis_index("c")
    sid = lax.axis_index("s")
    wid = sid * NC + c

    zero16 = jnp.zeros((16,), _f32)

    pltpu.sync_copy(bnd_hbm, bnd_v.at[pl.ds(0, 136)])
    for p in range(NPASS):
        my_base = PBASE[p] + RPT * wid

        def zrow(r, carry):
            for q in range(4):
                acc[r, pl.ds(16 * q, 16)] = zero16
            return carry
        lax.fori_loop(0, RPT, zrow, 0)

        bidx = p * NW + wid
        p0 = bnd_v[pl.ds(bidx, 16)][0]
        p1 = bnd_v[pl.ds(bidx + 1, 16)][0]
        k0 = lax.div(p0, BPS)
        k1 = lax.div(p1 + BPS - 1, BPS)

        def chunk_body(k, carry):
            r0 = k * BPS
            pltpu.sync_copy(ids_hbm.at[pl.ds(r0, BPS)], idv.at[pl.ds(0, BPS)])
            pltpu.sync_copy(pa_hbm.at[pl.ds(r0, BPS)], pa_v)

            def row_body(r, carry2):
                gp = r0 + r
                rid = idv[pl.ds(r, 16)][0]
                lid = rid - my_base

                @pl.when((gp >= p0) & (gp < p1))
                def _():
                    for q in range(4):
                        acc[lid, pl.ds(16 * q, 16)] = (
                            acc[lid, pl.ds(16 * q, 16)]
                            + pa_v[r, pl.ds(16 * q, 16)])
                return carry2
            lax.fori_loop(0, BPS, row_body, 0)
            return carry
        lax.fori_loop(k0, k1, chunk_body, 0)

        pl.when(wid < NW - 1)(
            lambda my_base=my_base: pltpu.sync_copy(
                acc, seg_hbm.at[pl.ds(my_base, RPT)]))
        pl.when(wid == NW - 1)(
            lambda my_base=my_base, lr=LROWS[p]: pltpu.sync_copy(
                acc.at[pl.ds(0, lr)], seg_hbm.at[pl.ds(my_base, lr)]))

    nb0 = (wid * NBG) // NW
    nb1 = ((wid + 1) * NBG) // NW

    def gbody(b, carry):
        r0 = b * BPG
        pltpu.sync_copy(ii_hbm.at[pl.ds(r0, BPG)], bi_v)
        pltpu.sync_copy(jj_hbm.at[pl.ds(r0, BPG)], bj_v)
        cp1 = pltpu.async_copy(t_hbm.at[bi_v], ti_v, sem1)
        cp2 = pltpu.async_copy(t_hbm.at[bj_v], tj_v, sem2)
        cp1.wait()
        cp2.wait()

        def rbody(r, rcarry):
            for q in range(4):
                u_i = ti_v[r, pl.ds(16 * q, 16)]
                v_i = ti_v[r, pl.ds(64 + 16 * q, 16)]
                u_j = tj_v[r, pl.ds(16 * q, 16)]
                v_j = tj_v[r, pl.ds(64 + 16 * q, 16)]
                x1 = jnp.maximum(u_i + v_j, 0.0)
                x2 = jnp.maximum(u_j + v_i, 0.0)
                s_v[r, pl.ds(16 * q, 16)] = x1 + x2
            return rcarry
        lax.fori_loop(0, BPG, rbody, 0)
        pltpu.sync_copy(s_v, s_hbm.at[pl.ds(r0, BPG)])
        return carry

    lax.fori_loop(nb0, nb1, gbody, 0)


def kernel(atom_features, pair_features, pair_split, pair_membership,
           atom_split, atom_to_pair, W_AA, b_AA, W_PA, b_PA, W_A, b_A,
           W_AP, b_AP, W_PP, b_PP, W_P, b_P):
    af = atom_features
    pf = pair_features

    w_uv = (jnp.zeros((DA, 128), _f32)
            .at[:, :H].set(W_AP[:DA])
            .at[:, 64:64 + H].set(W_AP[DA:]))
    bvec = jnp.zeros((1, 128), _f32).at[0, 64:64 + H].set(b_AP)
    wpa_pad = jnp.zeros((DP, 64), _f32).at[:, :H].set(W_PA)
    bpa_pad = jnp.zeros((1, 64), _f32).at[0, :H].set(b_PA)
    wp1_pad = jnp.zeros((64, H), _f32).at[:H].set(W_P[:H])
    wp2 = W_P[H:]
    wseg_pad = jnp.zeros((64, H), _f32).at[:H].set(W_A[H:])
    ii = atom_to_pair[:, 0]
    jj = atom_to_pair[:, 1]
    bases = jnp.array([PBASE[p] + RPT * t for p in range(NPASS)
                       for t in range(NW)], dtype=jnp.int32)
    bnd = jnp.searchsorted(pair_split, bases, side='left').astype(jnp.int32)
    bnd = jnp.concatenate([bnd, jnp.full((8,), NP, jnp.int32)])

    t_arr, aa = pl.pallas_call(
        _atoms_body,
        grid=(NAB,),
        in_specs=[
            pl.BlockSpec((BA, DA), lambda i: (i, 0)),
            pl.BlockSpec((DA, 128), lambda i: (0, 0)),
            pl.BlockSpec((1, 128), lambda i: (0, 0)),
            pl.BlockSpec((DA, H), lambda i: (0, 0)),
            pl.BlockSpec((1, H), lambda i: (0, 0)),
        ],
        out_specs=[
            pl.BlockSpec((BA, 128), lambda i: (i, 0)),
            pl.BlockSpec((BA, H), lambda i: (i, 0)),
        ],
        out_shape=[
            jax.ShapeDtypeStruct((NA, 128), _f32),
            jax.ShapeDtypeStruct((NA, H), _f32),
        ],
    )(af, w_uv, bvec, W_AA, b_AA.reshape(1, H))

    pa_arr = pl.pallas_call(
        _pa_body,
        grid=(NPB,),
        in_specs=[
            pl.BlockSpec((BPR, DP), lambda i: (i, 0)),
            pl.BlockSpec((DP, 64), lambda i: (0, 0)),
            pl.BlockSpec((1, 64), lambda i: (0, 0)),
        ],
        out_specs=pl.BlockSpec((BPR, 64), lambda i: (i, 0)),
        out_shape=jax.ShapeDtypeStruct((NP, 64), _f32),
    )(pf, wpa_pad, bpa_pad)

    s_arr, seg = _sc_stage(t_arr, ii, jj, pa_arr, pair_split, bnd)

    p_out = pl.pallas_call(
        _p_body,
        grid=(NPB,),
        in_specs=[
            pl.BlockSpec((BPR, 64), lambda i: (i, 0)),
            pl.BlockSpec((BPR, DP), lambda i: (i, 0)),
            pl.BlockSpec((64, H), lambda i: (0, 0)),
            pl.BlockSpec((DP, H), lambda i: (0, 0)),
            pl.BlockSpec((1, H), lambda i: (0, 0)),
            pl.BlockSpec((H, H), lambda i: (0, 0)),
            pl.BlockSpec((1, H), lambda i: (0, 0)),
        ],
        out_specs=pl.BlockSpec((BPR, H), lambda i: (i, 0)),
        out_shape=jax.ShapeDtypeStruct((NP, H), _f32),
    )(s_arr, pf, wp1_pad, W_PP, b_PP.reshape(1, H), wp2, b_P.reshape(1, H))

    a_out = pl.pallas_call(
        _a_body,
        grid=(NAB,),
        in_specs=[
            pl.BlockSpec((BA, H), lambda i: (i, 0)),
            pl.BlockSpec((BA, 64), lambda i: (i, 0)),
            pl.BlockSpec((H, H), lambda i: (0, 0)),
            pl.BlockSpec((64, H), lambda i: (0, 0)),
            pl.BlockSpec((1, H), lambda i: (0, 0)),
        ],
        out_specs=pl.BlockSpec((BA, H), lambda i: (i, 0)),
        out_shape=jax.ShapeDtypeStruct((NA, H), _f32),
    )(aa, seg, W_A[:H], wseg_pad, b_A.reshape(1, H))

    return (a_out, p_out)
